# Initial kernel scaffold; baseline (speedup 1.0000x reference)
import jax
import jax.numpy as jnp
from jax.experimental import pallas as pl


def kernel(action_pred_logits, human_pred_logits, object_pred_logits, human_pred_boxes, object_pred_boxes, image_sizes):
    raise NotImplementedError("write your pallas kernel here")


# speedup vs baseline: 1.1588x; 1.1588x over previous
import jax
import jax.numpy as jnp
from jax import lax
from jax.experimental import pallas as pl

B, Q = 200, 100
NUM_ACT = 44
NUM_OBJ = 112
TOP_K = 35
THRESH = 0.6
BB = 8
FLAT = Q * NUM_ACT


def _body(act_ref, hum_ref, obj_ref, hb_ref, ob_ref, sz_ref, out_ref):
    act = act_ref[...]
    hum = hum_ref[...]
    obj = obj_ref[...]

    def softmax_drop(x):
        m = jnp.max(x, axis=-1, keepdims=True)
        e = jnp.exp(x - m)
        return e[..., :-1] / jnp.sum(e, axis=-1, keepdims=True)

    act_cls = softmax_drop(act)
    hum_cls = softmax_drop(hum)
    obj_cls = softmax_drop(obj)

    keep = (jnp.max(act_cls, -1) > THRESH)
    keep &= jnp.max(hum_cls, -1) > THRESH
    keep &= jnp.max(obj_cls, -1) > THRESH

    scores = jnp.where(keep[..., None], act_cls, 0.0).reshape(BB, FLAT)
    iota = lax.broadcasted_iota(jnp.int32, (BB, FLAT), 1)

    vals, idxs = [], []
    s = scores
    for _ in range(TOP_K):
        m = jnp.max(s, axis=-1)
        eq = s == m[:, None]
        idx = jnp.min(jnp.where(eq, iota, FLAT), axis=-1)
        vals.append(m)
        idxs.append(idx)
        s = jnp.where(iota == idx[:, None], -jnp.inf, s)
    i_cls = jnp.stack(vals, axis=-1)
    topk_idx = jnp.stack(idxs, axis=-1)
    idx_box = topk_idx // NUM_ACT
    idx_act = topk_idx % NUM_ACT

    h_val = jnp.max(hum_cls, -1)
    o_val = jnp.max(obj_cls, -1)
    o_id = jnp.argmax(obj_cls, -1).astype(jnp.float32)

    sz = sz_ref[...]
    hh = sz[:, 0:1]
    ww = sz[:, 1:2]

    def to_xyxy(box):
        cx = box[..., 0] * ww
        cy = box[..., 1] * hh
        w = box[..., 2] * ww
        h = box[..., 3] * hh
        return [cx - 0.5 * w, cy - 0.5 * h, cx + 0.5 * w, cy + 0.5 * h]

    h_box = to_xyxy(hb_ref[...])
    o_box = to_xyxy(ob_ref[...])

    qiota = lax.broadcasted_iota(jnp.int32, (BB, TOP_K, Q), 2)
    eqq = idx_box[:, :, None] == qiota

    def gather_q(v):
        return jnp.max(jnp.where(eqq, v[:, None, :], -jnp.inf), axis=-1)

    chans = [gather_q(c) for c in h_box] + [gather_q(c) for c in o_box]
    h_cls = gather_q(h_val)
    o_cls = gather_q(o_val)
    o_sel = gather_q(o_id)

    valid = (i_cls > THRESH) & (h_cls > THRESH) & (o_cls > THRESH)
    score = h_cls * o_cls * i_cls
    pair = idx_act.astype(jnp.float32) * NUM_OBJ + o_sel

    eqp = pair[:, :, None] == pair[:, None, :]
    segmax = jnp.max(jnp.where(eqp, score[:, None, :], -jnp.inf), axis=-1)
    keep_nms = score >= segmax
    final = jnp.where(valid & keep_nms, score, 0.0)

    out = jnp.stack(chans + [h_cls, o_cls, final], axis=-1)
    out_ref[...] = out


def kernel(action_pred_logits, human_pred_logits, object_pred_logits,
           human_pred_boxes, object_pred_boxes, image_sizes):
    grid = (B // BB,)
    return pl.pallas_call(
        _body,
        grid=grid,
        in_specs=[
            pl.BlockSpec((BB, Q, NUM_ACT + 1), lambda i: (i, 0, 0)),
            pl.BlockSpec((BB, Q, 3), lambda i: (i, 0, 0)),
            pl.BlockSpec((BB, Q, NUM_OBJ + 1), lambda i: (i, 0, 0)),
            pl.BlockSpec((BB, Q, 4), lambda i: (i, 0, 0)),
            pl.BlockSpec((BB, Q, 4), lambda i: (i, 0, 0)),
            pl.BlockSpec((BB, 2), lambda i: (i, 0)),
        ],
        out_specs=pl.BlockSpec((BB, TOP_K, 11), lambda i: (i, 0, 0)),
        out_shape=jax.ShapeDtypeStruct((B, TOP_K, 11), jnp.float32),
    )(action_pred_logits, human_pred_logits, object_pred_logits,
      human_pred_boxes, object_pred_boxes, image_sizes)


# speedup vs baseline: 2.2544x; 1.9455x over previous
import jax
import jax.numpy as jnp
from jax import lax
from jax.experimental import pallas as pl
from jax.experimental.pallas import tpu as pltpu
from jax.experimental.pallas import tpu_sc as plsc

B, Q = 200, 100
NUM_ACT = 44
NUM_OBJ = 112
TOP_K = 35
THRESH = 0.6

L = 16
NW = 32
NBATCH_PER_W = (B + NW - 1) // NW
NGRP = (Q + L - 1) // L
FLAT = Q * NUM_ACT
NROW = FLAT // L
NROW_PAD = 288
NG2 = NROW_PAD // L
QPAD = 112
NCHAN = 11
OUT_ROW = 392
BIG = 1 << 30
NEG_INF = float("-inf")


def _iota():
    return lax.iota(jnp.int32, L)


def _splat_f(x):
    return jnp.broadcast_to(x.astype(jnp.float32) if hasattr(x, "astype") else jnp.float32(x), (L,))


def _splat_i(x):
    return jnp.broadcast_to(jnp.asarray(x, jnp.int32), (L,))


def _body(act_hbm, hum_hbm, obj_hbm, hb_hbm, ob_hbm, sz_hbm, out_hbm,
          act_v, hum_v, obj_v, hb_v, ob_v, sz_v, scores, grpmax, qv,
          topv, topf, pair_s, score_s, out_v):
    wid = lax.axis_index("s") * 2 + lax.axis_index("c")
    pltpu.sync_copy(sz_hbm, sz_v)
    lane = _iota()
    lane0 = lane == 0

    def do_batch(b):
        pltpu.sync_copy(act_hbm.at[b], act_v)
        pltpu.sync_copy(hum_hbm.at[b], hum_v)
        pltpu.sync_copy(obj_hbm.at[b], obj_v)
        pltpu.sync_copy(hb_hbm.at[b], hb_v)
        pltpu.sync_copy(ob_hbm.at[b], ob_v)

        hh = plsc.load_gather(sz_v, [_splat_i(b * 2)])
        ww = plsc.load_gather(sz_v, [_splat_i(b * 2 + 1)])

        def init_pad(r, _):
            plsc.store_scatter(scores, [r * L + lane], _splat_f(NEG_INF))
            return 0
        lax.fori_loop(NROW, NROW_PAD, init_pad, 0)

        def do_group(g, _):
            ql = g * L + lane
            qmask = ql < Q
            qc = jnp.minimum(ql, Q - 1)
            qa = qc * (NUM_ACT + 1)
            qo = qc * (NUM_OBJ + 1)

            def amax_step(c0, m44):
                for j in range(11):
                    v = plsc.load_gather(act_v, [qa + _splat_i(c0 * 11 + j)])
                    m44 = jnp.maximum(m44, v)
                return m44
            m44 = lax.fori_loop(0, 4, amax_step, _splat_f(NEG_INF))
            v44 = plsc.load_gather(act_v, [qa + NUM_ACT])
            ma = jnp.maximum(m44, v44)

            def asum_step(c0, s):
                for j in range(9):
                    v = plsc.load_gather(act_v, [qa + c0 * 9 + j])
                    s = s + jnp.exp(v - ma)
                return s
            sa = lax.fori_loop(0, 5, asum_step, jnp.zeros((L,), jnp.float32))
            p_act = jnp.exp(m44 - ma) / sa

            qh = qc * 3
            h0 = plsc.load_gather(hum_v, [qh])
            h1 = plsc.load_gather(hum_v, [qh + 1])
            h2 = plsc.load_gather(hum_v, [qh + 2])
            mh = jnp.maximum(jnp.maximum(h0, h1), h2)
            sh = jnp.exp(h0 - mh) + jnp.exp(h1 - mh) + jnp.exp(h2 - mh)
            h_val = jnp.exp(jnp.maximum(h0, h1) - mh) / sh

            def omax_step(c0, carry):
                mo, ido = carry
                for j in range(16):
                    c = c0 * 16 + j
                    v = plsc.load_gather(obj_v, [qo + c])
                    gt = v > mo
                    ido = jnp.where(gt, _splat_i(c), ido)
                    mo = jnp.maximum(mo, v)
                return mo, ido
            mo, ido = lax.fori_loop(0, 7, omax_step,
                                    (_splat_f(NEG_INF), _splat_i(0)))
            v112 = plsc.load_gather(obj_v, [qo + NUM_OBJ])
            mob = jnp.maximum(mo, v112)

            def osum_step(c0, s):
                for j in range(16):
                    v = plsc.load_gather(obj_v, [qo + c0 * 16 + j])
                    s = s + jnp.exp(v - mob)
                return s
            so = lax.fori_loop(0, 7, osum_step, jnp.zeros((L,), jnp.float32))
            so = so + jnp.exp(v112 - mob)
            o_val = jnp.exp(mo - mob) / so

            keep = (p_act > THRESH) & (h_val > THRESH) & (o_val > THRESH)

            fbase = qc * NUM_ACT
            def astore_step(c0, _):
                for j in range(11):
                    c = c0 * 11 + j
                    v = plsc.load_gather(act_v, [qa + c])
                    sc = jnp.where(keep, jnp.exp(v - ma) / sa, 0.0)
                    plsc.store_scatter(scores, [fbase + c], sc, mask=qmask)
                return 0
            lax.fori_loop(0, 4, astore_step, 0)

            def put(ch, v):
                plsc.store_scatter(qv, [ch * QPAD + ql], v, mask=qmask)

            for ref, base in ((hb_v, 0), (ob_v, 4)):
                qb = qc * 4
                cx = plsc.load_gather(ref, [qb]) * ww
                cy = plsc.load_gather(ref, [qb + 1]) * hh
                w2 = plsc.load_gather(ref, [qb + 2]) * ww * 0.5
                h2b = plsc.load_gather(ref, [qb + 3]) * hh * 0.5
                put(base + 0, cx - w2)
                put(base + 1, cy - h2b)
                put(base + 2, cx + w2)
                put(base + 3, cy + h2b)
            put(8, h_val)
            put(9, o_val)
            put(10, ido.astype(jnp.float32))
            return 0
        lax.fori_loop(0, NGRP, do_group, 0)

        def build_g2(g2, _):
            m = _splat_f(NEG_INF)
            def rstep(rr, m):
                for j in range(4):
                    r = g2 * L + rr * 4 + j
                    m = jnp.maximum(m, plsc.load_gather(scores, [r * L + lane]))
                return m
            m = lax.fori_loop(0, 4, rstep, m)
            plsc.store_scatter(grpmax, [g2 * L + lane], m)
            return 0
        lax.fori_loop(0, NG2, build_g2, 0)

        def topk_step(k, _):
            def gm_step(j, m):
                for t in range(3):
                    m = jnp.maximum(m, plsc.load_gather(
                        grpmax, [(j * 3 + t) * L + lane]))
                return m
            m = lax.fori_loop(0, 6, gm_step, _splat_f(NEG_INF))
            gmax = jnp.max(m)

            def gf_step(j, gf):
                for t in range(3):
                    g2 = j * 3 + t
                    v = plsc.load_gather(grpmax, [g2 * L + lane])
                    gf = jnp.minimum(gf, jnp.where(v == gmax, _splat_i(g2), BIG))
                return gf
            gf = lax.fori_loop(0, 6, gf_step, _splat_i(BIG))
            gsel = jnp.min(gf)

            def rf_step(rr, rf):
                for t in range(4):
                    r = gsel * L + rr * 4 + t
                    v = plsc.load_gather(scores, [r * L + lane])
                    rf = jnp.minimum(rf, jnp.where(v == gmax, r, BIG))
                return rf
            rf = lax.fori_loop(0, 4, rf_step, _splat_i(BIG))
            rsel = jnp.min(rf)

            vrow = plsc.load_gather(scores, [rsel * L + lane])
            lsel = plsc.all_reduce_ffs(vrow == gmax)
            fsel = rsel * L + jnp.max(lsel)

            plsc.store_scatter(topv, [_splat_i(k)], _splat_f(gmax), mask=lane0)
            plsc.store_scatter(topf, [_splat_i(k)], _splat_i(fsel), mask=lane0)
            plsc.store_scatter(scores, [_splat_i(fsel)], _splat_f(NEG_INF),
                               mask=lane0)

            def upd_step(rr, m):
                for t in range(4):
                    r = gsel * L + rr * 4 + t
                    m = jnp.maximum(m, plsc.load_gather(scores, [r * L + lane]))
                return m
            m2 = lax.fori_loop(0, 4, upd_step, _splat_f(NEG_INF))
            plsc.store_scatter(grpmax, [gsel * L + lane], m2)
            return 0
        lax.fori_loop(0, TOP_K, topk_step, 0)

        ch_cache = []
        for t in range(3):
            kl = t * L + lane
            kmask = kl < TOP_K
            kc = jnp.minimum(kl, TOP_K - 1)
            f = plsc.load_gather(topf, [kc])
            iv = plsc.load_gather(topv, [kc])
            idx_box = f // NUM_ACT
            idx_act = f - idx_box * NUM_ACT
            chans = [plsc.load_gather(qv, [c * QPAD + idx_box])
                     for c in range(NCHAN)]
            h_cls = chans[8]
            o_cls = chans[9]
            o_ii = chans[10].astype(jnp.int32)
            pair = idx_act * NUM_OBJ + o_ii
            score = h_cls * o_cls * iv
            plsc.store_scatter(pair_s, [kl], jnp.where(kmask, pair, -1))
            plsc.store_scatter(score_s, [kl], score)
            ch_cache.append((kl, kmask, iv, chans, pair, score))

        for t in range(3):
            kl, kmask, iv, chans, pair, score = ch_cache[t]
            def nms_step(j, sm):
                pj = plsc.load_gather(pair_s, [_splat_i(j)])
                sj = plsc.load_gather(score_s, [_splat_i(j)])
                return jnp.where(pair == pj, jnp.maximum(sm, sj), sm)
            segmax = lax.fori_loop(0, TOP_K, nms_step, _splat_f(NEG_INF))
            h_cls = chans[8]
            o_cls = chans[9]
            valid = (iv > THRESH) & (h_cls > THRESH) & (o_cls > THRESH)
            final = jnp.where(valid & (score >= segmax), score, 0.0)
            outch = chans[:8] + [h_cls, o_cls, final]
            for c in range(NCHAN):
                plsc.store_scatter(out_v, [kl * NCHAN + c], outch[c],
                                   mask=kmask)

        pltpu.sync_copy(out_v, out_hbm.at[b])

    def batch_loop(i, _):
        b = wid + NW * i
        @pl.when(b < B)
        def _():
            do_batch(b)
        return 0
    lax.fori_loop(0, NBATCH_PER_W, batch_loop, 0)


def kernel(action_pred_logits, human_pred_logits, object_pred_logits,
           human_pred_boxes, object_pred_boxes, image_sizes):
    mesh = plsc.VectorSubcoreMesh(core_axis_name="c", subcore_axis_name="s",
                                  num_cores=2, num_subcores=16)
    f32 = jnp.float32
    out = pl.kernel(
        _body,
        out_type=jax.ShapeDtypeStruct((B, OUT_ROW), f32),
        mesh=mesh,
        compiler_params=pltpu.CompilerParams(needs_layout_passes=False),
        scratch_types=[
            pltpu.VMEM((Q * (NUM_ACT + 1),), f32),
            pltpu.VMEM((Q * 3,), f32),
            pltpu.VMEM((Q * (NUM_OBJ + 1),), f32),
            pltpu.VMEM((Q * 4,), f32),
            pltpu.VMEM((Q * 4,), f32),
            pltpu.VMEM((B * 2,), f32),
            pltpu.VMEM((NROW_PAD * L,), f32),
            pltpu.VMEM((NROW_PAD,), f32),
            pltpu.VMEM((NCHAN * QPAD,), f32),
            pltpu.VMEM((3 * L,), f32),
            pltpu.VMEM((3 * L,), jnp.int32),
            pltpu.VMEM((3 * L,), jnp.int32),
            pltpu.VMEM((3 * L,), f32),
            pltpu.VMEM((OUT_ROW,), f32),
        ],
    )(action_pred_logits.reshape(B, -1), human_pred_logits.reshape(B, -1),
      object_pred_logits.reshape(B, -1), human_pred_boxes.reshape(B, -1),
      object_pred_boxes.reshape(B, -1), image_sizes.reshape(-1))
    return out[:, :TOP_K * NCHAN].reshape(B, TOP_K, NCHAN)


# speedup vs baseline: 4.6791x; 2.0755x over previous
import jax
import jax.numpy as jnp
from jax import lax
from jax.experimental import pallas as pl
from jax.experimental.pallas import tpu as pltpu
from jax.experimental.pallas import tpu_sc as plsc

B, Q = 200, 100
NUM_ACT = 44
NUM_OBJ = 112
TOP_K = 35
THRESH = 0.6

L = 16
NW = 32
NBATCH_PER_W = (B + NW - 1) // NW
NGRP = (Q + L - 1) // L
FLAT = Q * NUM_ACT
NROW = FLAT // L
NROW_PAD = 288
NG2 = NROW_PAD // L
QPAD = 112
NCHAN = 11
OUT_ROW = 392
BIG = 1 << 30
NEG_INF = float("-inf")


def _iota():
    return lax.iota(jnp.int32, L)


def _splat_f(x):
    return jnp.broadcast_to(x.astype(jnp.float32) if hasattr(x, "astype") else jnp.float32(x), (L,))


def _splat_i(x):
    return jnp.broadcast_to(jnp.asarray(x, jnp.int32), (L,))


def _obj_stats(obj_v, qc):
    qo = qc * (NUM_OBJ + 1)

    def omax_step(c0, carry):
        mo, ido = carry
        for j in range(16):
            c = c0 * 16 + j
            v = plsc.load_gather(obj_v, [qo + c])
            gt = v > mo
            ido = jnp.where(gt, _splat_i(c), ido)
            mo = jnp.maximum(mo, v)
        return mo, ido
    mo, ido = lax.fori_loop(0, 7, omax_step, (_splat_f(NEG_INF), _splat_i(0)))
    v112 = plsc.load_gather(obj_v, [qo + NUM_OBJ])
    mob = jnp.maximum(mo, v112)

    def osum_step(c0, ss):
        ss = list(ss)
        for j in range(16):
            v = plsc.load_gather(obj_v, [qo + c0 * 16 + j])
            ss[j % 4] = ss[j % 4] + jnp.exp(v - mob)
        return tuple(ss)
    zo = jnp.zeros((L,), jnp.float32)
    t0, t1, t2, t3 = lax.fori_loop(0, 7, osum_step, (zo, zo, zo, zo))
    so = ((t0 + t1) + (t2 + t3)) + jnp.exp(v112 - mob)
    return jnp.exp(mo - mob) / so, ido


def _body(act_hbm, hum_hbm, obj_hbm, hb_hbm, ob_hbm, sz_hbm, out_hbm,
          act_v, hum_v, obj_v, hb_v, ob_v, sz_v, scores, grpmax, qv,
          topv, topf, pair_s, score_s, anyk, out_v, sems):
    wid = lax.axis_index("s") * 2 + lax.axis_index("c")
    pltpu.sync_copy(sz_hbm, sz_v)
    lane = _iota()
    lane0 = lane == 0

    def init_pad(r, _):
        plsc.store_scatter(scores, [r * L + lane], _splat_f(NEG_INF))
        return 0
    lax.fori_loop(NROW, NROW_PAD, init_pad, 0)

    def do_batch(b):
        cps = [pltpu.async_copy(act_hbm.at[b], act_v, sems.at[0]),
               pltpu.async_copy(hum_hbm.at[b], hum_v, sems.at[1]),
               pltpu.async_copy(obj_hbm.at[b], obj_v, sems.at[2]),
               pltpu.async_copy(hb_hbm.at[b], hb_v, sems.at[3]),
               pltpu.async_copy(ob_hbm.at[b], ob_v, sems.at[4])]
        for cp in cps:
            cp.wait()

        hh = plsc.load_gather(sz_v, [_splat_i(b * 2)])
        ww = plsc.load_gather(sz_v, [_splat_i(b * 2 + 1)])
        anyk[...] = jnp.zeros((L,), jnp.int32)

        def do_group(g, _):
            ql = g * L + lane
            qmask = ql < Q
            qc = jnp.minimum(ql, Q - 1)
            qa = qc * (NUM_ACT + 1)

            def amax_step(c0, mm):
                a, bm = mm
                for j in range(11):
                    v = plsc.load_gather(act_v, [qa + _splat_i(c0 * 11 + j)])
                    if j % 2 == 0:
                        a = jnp.maximum(a, v)
                    else:
                        bm = jnp.maximum(bm, v)
                return a, bm
            ninf = _splat_f(NEG_INF)
            ma0, ma1 = lax.fori_loop(0, 4, amax_step, (ninf, ninf))
            m44 = jnp.maximum(ma0, ma1)
            v44 = plsc.load_gather(act_v, [qa + NUM_ACT])
            ma = jnp.maximum(m44, v44)

            def asum_step(c0, ss):
                ss = list(ss)
                for j in range(9):
                    v = plsc.load_gather(act_v, [qa + c0 * 9 + j])
                    ss[j % 3] = ss[j % 3] + jnp.exp(v - ma)
                return tuple(ss)
            z = jnp.zeros((L,), jnp.float32)
            s0, s1, s2 = lax.fori_loop(0, 5, asum_step, (z, z, z))
            sa = (s0 + s1) + s2
            p_act = jnp.exp(m44 - ma) / sa

            qh = qc * 3
            h0 = plsc.load_gather(hum_v, [qh])
            h1 = plsc.load_gather(hum_v, [qh + 1])
            h2 = plsc.load_gather(hum_v, [qh + 2])
            mh = jnp.maximum(jnp.maximum(h0, h1), h2)
            sh = jnp.exp(h0 - mh) + jnp.exp(h1 - mh) + jnp.exp(h2 - mh)
            h_val = jnp.exp(jnp.maximum(h0, h1) - mh) / sh

            pre = (p_act > THRESH) & (h_val > THRESH) & qmask
            plsc.store_scatter(qv, [ql], h_val, mask=qmask)
            plsc.store_scatter(qv, [QPAD + ql], jnp.zeros((L,), jnp.float32),
                               mask=qmask)
            plsc.store_scatter(qv, [2 * QPAD + ql], ma, mask=qmask)

            @pl.when(jnp.max(pre.astype(jnp.int32)) > 0)
            def _():
                o_val, _ido = _obj_stats(obj_v, qc)
                keep = pre & (o_val > THRESH)
                factor = jnp.where(keep, 1.0 / sa, 0.0)
                plsc.store_scatter(qv, [QPAD + ql], factor, mask=qmask)
                anyk[...] = anyk[...] | keep.astype(jnp.int32)
            return 0
        lax.fori_loop(0, NGRP, do_group, 0)

        any_keep = jnp.max(anyk[...]) > 0

        @pl.when(jnp.logical_not(any_keep))
        def _():
            for t in range(3):
                kl = t * L + lane
                kmask = kl < TOP_K
                plsc.store_scatter(topv, [kl], jnp.zeros((L,), jnp.float32),
                                   mask=kmask)
                plsc.store_scatter(topf, [kl], kl, mask=kmask)

        @pl.when(any_keep)
        def _():
            def fill_group(g, _):
                ql = g * L + lane
                qmask = ql < Q
                qc = jnp.minimum(ql, Q - 1)
                qa = qc * (NUM_ACT + 1)
                factor = plsc.load_gather(qv, [QPAD + qc])
                ma = plsc.load_gather(qv, [2 * QPAD + qc])
                fbase = qc * NUM_ACT

                def astore_step(c0, _):
                    for j in range(11):
                        c = c0 * 11 + j
                        v = plsc.load_gather(act_v, [qa + c])
                        sc = jnp.exp(v - ma) * factor
                        plsc.store_scatter(scores, [fbase + c], sc, mask=qmask)
                    return 0
                lax.fori_loop(0, 4, astore_step, 0)
                return 0
            lax.fori_loop(0, NGRP, fill_group, 0)

            def build_g2(g2, _):
                m = _splat_f(NEG_INF)
                def rstep(rr, m):
                    for j in range(4):
                        r = g2 * L + rr * 4 + j
                        m = jnp.maximum(
                            m, plsc.load_gather(scores, [r * L + lane]))
                    return m
                m = lax.fori_loop(0, 4, rstep, m)
                plsc.store_scatter(grpmax, [g2 * L + lane], m)
                return 0
            lax.fori_loop(0, NG2, build_g2, 0)

            def topk_step(k, _):
                def gm_step(j, carry):
                    m, gidx = carry
                    for t in range(3):
                        g2 = j * 3 + t
                        v = plsc.load_gather(grpmax, [g2 * L + lane])
                        gt = v > m
                        gidx = jnp.where(gt, _splat_i(g2), gidx)
                        m = jnp.maximum(m, v)
                    return m, gidx
                m, gidx = lax.fori_loop(0, 6, gm_step,
                                        (_splat_f(NEG_INF), _splat_i(0)))
                gmax = jnp.max(m)
                gsel = jnp.min(jnp.where(m == gmax, gidx, BIG))

                def rf_step(rr, rf):
                    for t in range(4):
                        r = gsel * L + rr * 4 + t
                        v = plsc.load_gather(scores, [r * L + lane])
                        rf = jnp.minimum(rf, jnp.where(v == gmax, r, BIG))
                    return rf
                rf = lax.fori_loop(0, 4, rf_step, _splat_i(BIG))
                rsel = jnp.min(rf)

                vrow = plsc.load_gather(scores, [rsel * L + lane])
                lsel = jnp.max(plsc.all_reduce_ffs(vrow == gmax))
                fsel = rsel * L + lsel

                plsc.store_scatter(topv, [_splat_i(k)], _splat_f(gmax),
                                   mask=lane0)
                plsc.store_scatter(topf, [_splat_i(k)], _splat_i(fsel),
                                   mask=lane0)
                plsc.store_scatter(scores, [_splat_i(fsel)], _splat_f(NEG_INF),
                                   mask=lane0)

                col = plsc.load_gather(scores, [(gsel * L + lane) * L + lsel])
                cm = jnp.max(col)
                plsc.store_scatter(grpmax, [_splat_i(gsel * L + lsel)],
                                   _splat_f(cm), mask=lane0)
                return 0
            lax.fori_loop(0, TOP_K, topk_step, 0)

        ch_cache = []
        for t in range(3):
            kl = t * L + lane
            kmask = kl < TOP_K
            kc = jnp.minimum(kl, TOP_K - 1)
            f = plsc.load_gather(topf, [kc])
            iv = plsc.load_gather(topv, [kc])
            idx_box = f // NUM_ACT
            idx_act = f - idx_box * NUM_ACT
            h_cls = plsc.load_gather(qv, [idx_box])
            o_cls, o_ii = _obj_stats(obj_v, idx_box)

            boxes = []
            for ref in (hb_v, ob_v):
                qb = idx_box * 4
                cx = plsc.load_gather(ref, [qb]) * ww
                cy = plsc.load_gather(ref, [qb + 1]) * hh
                w2 = plsc.load_gather(ref, [qb + 2]) * ww * 0.5
                h2b = plsc.load_gather(ref, [qb + 3]) * hh * 0.5
                boxes += [cx - w2, cy - h2b, cx + w2, cy + h2b]

            pair = idx_act * NUM_OBJ + o_ii
            score = h_cls * o_cls * iv
            plsc.store_scatter(pair_s, [kl], jnp.where(kmask, pair, -1))
            plsc.store_scatter(score_s, [kl], score)
            ch_cache.append((kl, kmask, iv, boxes, h_cls, o_cls, pair, score))

        segmaxes = [_splat_f(NEG_INF)] * 3
        for j in range(TOP_K):
            pj = plsc.load_gather(pair_s, [_splat_i(j)])
            sj = plsc.load_gather(score_s, [_splat_i(j)])
            for t in range(3):
                pr = ch_cache[t][6]
                segmaxes[t] = jnp.where(pr == pj,
                                        jnp.maximum(segmaxes[t], sj),
                                        segmaxes[t])
        for t in range(3):
            kl, kmask, iv, boxes, h_cls, o_cls, pair, score = ch_cache[t]
            valid = (iv > THRESH) & (h_cls > THRESH) & (o_cls > THRESH)
            final = jnp.where(valid & (score >= segmaxes[t]), score, 0.0)
            outch = boxes + [h_cls, o_cls, final]
            for c in range(NCHAN):
                plsc.store_scatter(out_v, [kl * NCHAN + c], outch[c],
                                   mask=kmask)

        pltpu.sync_copy(out_v, out_hbm.at[b])

    def batch_loop(i, _):
        b = wid + NW * i
        @pl.when(b < B)
        def _():
            do_batch(b)
        return 0
    lax.fori_loop(0, NBATCH_PER_W, batch_loop, 0)


def kernel(action_pred_logits, human_pred_logits, object_pred_logits,
           human_pred_boxes, object_pred_boxes, image_sizes):
    mesh = plsc.VectorSubcoreMesh(core_axis_name="c", subcore_axis_name="s",
                                  num_cores=2, num_subcores=16)
    f32 = jnp.float32
    out = pl.kernel(
        _body,
        out_type=jax.ShapeDtypeStruct((B, OUT_ROW), f32),
        mesh=mesh,
        compiler_params=pltpu.CompilerParams(needs_layout_passes=False),
        scratch_types=[
            pltpu.VMEM((Q * (NUM_ACT + 1),), f32),
            pltpu.VMEM((Q * 3,), f32),
            pltpu.VMEM((Q * (NUM_OBJ + 1),), f32),
            pltpu.VMEM((Q * 4,), f32),
            pltpu.VMEM((Q * 4,), f32),
            pltpu.VMEM((B * 2,), f32),
            pltpu.VMEM((NROW_PAD * L,), f32),
            pltpu.VMEM((NROW_PAD,), f32),
            pltpu.VMEM((3 * QPAD,), f32),
            pltpu.VMEM((3 * L,), f32),
            pltpu.VMEM((3 * L,), jnp.int32),
            pltpu.VMEM((3 * L,), jnp.int32),
            pltpu.VMEM((3 * L,), f32),
            pltpu.VMEM((L,), jnp.int32),
            pltpu.VMEM((OUT_ROW,), f32),
            pltpu.SemaphoreType.DMA((5,)),
        ],
    )(action_pred_logits.reshape(B, -1), human_pred_logits.reshape(B, -1),
      object_pred_logits.reshape(B, -1), human_pred_boxes.reshape(B, -1),
      object_pred_boxes.reshape(B, -1), image_sizes.reshape(-1))
    return out[:, :TOP_K * NCHAN].reshape(B, TOP_K, NCHAN)


# speedup vs baseline: 4.6932x; 1.0030x over previous
import jax
import jax.numpy as jnp
from jax import lax
from jax.experimental import pallas as pl
from jax.experimental.pallas import tpu as pltpu
from jax.experimental.pallas import tpu_sc as plsc

B, Q = 200, 100
NUM_ACT = 44
NUM_OBJ = 112
TOP_K = 35
THRESH = 0.6

L = 16
NW = 32
NBATCH_PER_W = (B + NW - 1) // NW
NGRP = (Q + L - 1) // L
FLAT = Q * NUM_ACT
NROW = FLAT // L
NROW_PAD = 288
NG2 = NROW_PAD // L
QPAD = 112
NCHAN = 11
OUT_ROW = 392
BIG = 1 << 30
NEG_INF = float("-inf")


def _iota():
    return lax.iota(jnp.int32, L)


def _splat_f(x):
    return jnp.broadcast_to(x.astype(jnp.float32) if hasattr(x, "astype") else jnp.float32(x), (L,))


def _splat_i(x):
    return jnp.broadcast_to(jnp.asarray(x, jnp.int32), (L,))


def _obj_stats(obj_v, qc):
    qo = qc * (NUM_OBJ + 1)

    def omax_step(c0, carry):
        mo, ido = carry
        for j in range(16):
            c = c0 * 16 + j
            v = plsc.load_gather(obj_v, [qo + c])
            gt = v > mo
            ido = jnp.where(gt, _splat_i(c), ido)
            mo = jnp.maximum(mo, v)
        return mo, ido
    mo, ido = lax.fori_loop(0, 7, omax_step, (_splat_f(NEG_INF), _splat_i(0)))
    v112 = plsc.load_gather(obj_v, [qo + NUM_OBJ])
    mob = jnp.maximum(mo, v112)

    def osum_step(c0, ss):
        ss = list(ss)
        for j in range(16):
            v = plsc.load_gather(obj_v, [qo + c0 * 16 + j])
            ss[j % 4] = ss[j % 4] + jnp.exp(v - mob)
        return tuple(ss)
    zo = jnp.zeros((L,), jnp.float32)
    t0, t1, t2, t3 = lax.fori_loop(0, 7, osum_step, (zo, zo, zo, zo))
    so = ((t0 + t1) + (t2 + t3)) + jnp.exp(v112 - mob)
    return jnp.exp(mo - mob) / so, ido


def _body(act_hbm, hum_hbm, obj_hbm, hb_hbm, ob_hbm, sz_hbm, out_hbm,
          act_v, hum_v, obj_v, hb_v, ob_v, sz_v, scores, grpmax, qv,
          topv, topf, pair_s, score_s, anyk, out_v, sems):
    wid = lax.axis_index("s") * 2 + lax.axis_index("c")
    pltpu.sync_copy(sz_hbm, sz_v)
    lane = _iota()
    lane0 = lane == 0

    def init_pad(r, _):
        plsc.store_scatter(scores, [r * L + lane], _splat_f(NEG_INF))
        return 0
    lax.fori_loop(NROW, NROW_PAD, init_pad, 0)

    def do_batch(b):
        cps = [pltpu.async_copy(act_hbm.at[b], act_v, sems.at[0]),
               pltpu.async_copy(hum_hbm.at[b], hum_v, sems.at[1]),
               pltpu.async_copy(obj_hbm.at[b], obj_v, sems.at[2]),
               pltpu.async_copy(hb_hbm.at[b], hb_v, sems.at[3]),
               pltpu.async_copy(ob_hbm.at[b], ob_v, sems.at[4])]
        for cp in cps:
            cp.wait()

        hh = plsc.load_gather(sz_v, [_splat_i(b * 2)])
        ww = plsc.load_gather(sz_v, [_splat_i(b * 2 + 1)])
        anyk[...] = jnp.zeros((L,), jnp.int32)

        def do_group(g, _):
            ql = g * L + lane
            qmask = ql < Q
            qc = jnp.minimum(ql, Q - 1)
            qa = qc * (NUM_ACT + 1)

            def amax_step(c0, mm):
                a, bm = mm
                for j in range(11):
                    v = plsc.load_gather(act_v, [qa + _splat_i(c0 * 11 + j)])
                    if j % 2 == 0:
                        a = jnp.maximum(a, v)
                    else:
                        bm = jnp.maximum(bm, v)
                return a, bm
            ninf = _splat_f(NEG_INF)
            ma0, ma1 = lax.fori_loop(0, 4, amax_step, (ninf, ninf))
            m44 = jnp.maximum(ma0, ma1)
            v44 = plsc.load_gather(act_v, [qa + NUM_ACT])
            ma = jnp.maximum(m44, v44)

            def asum_step(c0, ss):
                ss = list(ss)
                for j in range(9):
                    v = plsc.load_gather(act_v, [qa + c0 * 9 + j])
                    ss[j % 3] = ss[j % 3] + jnp.exp(v - ma)
                return tuple(ss)
            z = jnp.zeros((L,), jnp.float32)
            s0, s1, s2 = lax.fori_loop(0, 5, asum_step, (z, z, z))
            sa = (s0 + s1) + s2
            p_act = jnp.exp(m44 - ma) / sa

            qh = qc * 3
            h0 = plsc.load_gather(hum_v, [qh])
            h1 = plsc.load_gather(hum_v, [qh + 1])
            h2 = plsc.load_gather(hum_v, [qh + 2])
            mh = jnp.maximum(jnp.maximum(h0, h1), h2)
            sh = jnp.exp(h0 - mh) + jnp.exp(h1 - mh) + jnp.exp(h2 - mh)
            h_val = jnp.exp(jnp.maximum(h0, h1) - mh) / sh

            pre = (p_act > THRESH) & (h_val > THRESH) & qmask
            plsc.store_scatter(qv, [ql], h_val, mask=qmask)
            plsc.store_scatter(qv, [QPAD + ql], jnp.zeros((L,), jnp.float32),
                               mask=qmask)
            plsc.store_scatter(qv, [2 * QPAD + ql], ma, mask=qmask)

            @pl.when(jnp.max(pre.astype(jnp.int32)) > 0)
            def _():
                o_val, _ido = _obj_stats(obj_v, qc)
                keep = pre & (o_val > THRESH)
                factor = jnp.where(keep, 1.0 / sa, 0.0)
                plsc.store_scatter(qv, [QPAD + ql], factor, mask=qmask)
                anyk[...] = anyk[...] | keep.astype(jnp.int32)
            return 0
        lax.fori_loop(0, NGRP, do_group, 0)

        any_keep = jnp.max(anyk[...]) > 0

        @pl.when(jnp.logical_not(any_keep))
        def _():
            for t in range(3):
                kl = t * L + lane
                kmask = kl < TOP_K
                plsc.store_scatter(topv, [kl], jnp.zeros((L,), jnp.float32),
                                   mask=kmask)
                plsc.store_scatter(topf, [kl], kl, mask=kmask)

        @pl.when(any_keep)
        def _():
            def fill_group(g, _):
                ql = g * L + lane
                qmask = ql < Q
                qc = jnp.minimum(ql, Q - 1)
                qa = qc * (NUM_ACT + 1)
                factor = plsc.load_gather(qv, [QPAD + qc])
                ma = plsc.load_gather(qv, [2 * QPAD + qc])
                fbase = qc * NUM_ACT

                def astore_step(c0, _):
                    for j in range(11):
                        c = c0 * 11 + j
                        v = plsc.load_gather(act_v, [qa + c])
                        sc = jnp.exp(v - ma) * factor
                        plsc.store_scatter(scores, [fbase + c], sc, mask=qmask)
                    return 0
                lax.fori_loop(0, 4, astore_step, 0)
                return 0
            lax.fori_loop(0, NGRP, fill_group, 0)

            def build_g2(g2, _):
                m = _splat_f(NEG_INF)
                def rstep(rr, m):
                    for j in range(4):
                        r = g2 * L + rr * 4 + j
                        m = jnp.maximum(
                            m, plsc.load_gather(scores, [r * L + lane]))
                    return m
                m = lax.fori_loop(0, 4, rstep, m)
                plsc.store_scatter(grpmax, [g2 * L + lane], m)
                return 0
            lax.fori_loop(0, NG2, build_g2, 0)

            def topk_step(k, _):
                def gm_step(j, carry):
                    m, gidx = carry
                    for t in range(3):
                        g2 = j * 3 + t
                        v = plsc.load_gather(grpmax, [g2 * L + lane])
                        gt = v > m
                        gidx = jnp.where(gt, _splat_i(g2), gidx)
                        m = jnp.maximum(m, v)
                    return m, gidx
                m, gidx = lax.fori_loop(0, 6, gm_step,
                                        (_splat_f(NEG_INF), _splat_i(0)))
                gmax = jnp.max(m)
                gsel = jnp.min(jnp.where(m == gmax, gidx, BIG))

                def rf_step(rr, rf):
                    for t in range(4):
                        r = gsel * L + rr * 4 + t
                        v = plsc.load_gather(scores, [r * L + lane])
                        rf = jnp.minimum(rf, jnp.where(v == gmax, r, BIG))
                    return rf
                rf = lax.fori_loop(0, 4, rf_step, _splat_i(BIG))
                rsel = jnp.min(rf)

                vrow = plsc.load_gather(scores, [rsel * L + lane])
                lsel = jnp.max(plsc.all_reduce_ffs(vrow == gmax))
                fsel = rsel * L + lsel

                plsc.store_scatter(topv, [_splat_i(k)], _splat_f(gmax),
                                   mask=lane0)
                plsc.store_scatter(topf, [_splat_i(k)], _splat_i(fsel),
                                   mask=lane0)
                plsc.store_scatter(scores, [_splat_i(fsel)], _splat_f(NEG_INF),
                                   mask=lane0)

                col = plsc.load_gather(scores, [(gsel * L + lane) * L + lsel])
                cm = jnp.max(col)
                plsc.store_scatter(grpmax, [_splat_i(gsel * L + lsel)],
                                   _splat_f(cm), mask=lane0)
                return 0
            lax.fori_loop(0, TOP_K, topk_step, 0)

        ch_cache = []
        for t in range(3):
            kl = t * L + lane
            kmask = kl < TOP_K
            kc = jnp.minimum(kl, TOP_K - 1)
            f = plsc.load_gather(topf, [kc])
            iv = plsc.load_gather(topv, [kc])
            idx_box = f // NUM_ACT
            idx_act = f - idx_box * NUM_ACT
            h_cls = plsc.load_gather(qv, [idx_box])
            o_cls, o_ii = _obj_stats(obj_v, idx_box)

            boxes = []
            for ref in (hb_v, ob_v):
                qb = idx_box * 4
                cx = plsc.load_gather(ref, [qb]) * ww
                cy = plsc.load_gather(ref, [qb + 1]) * hh
                w2 = plsc.load_gather(ref, [qb + 2]) * ww * 0.5
                h2b = plsc.load_gather(ref, [qb + 3]) * hh * 0.5
                boxes += [cx - w2, cy - h2b, cx + w2, cy + h2b]

            pair = idx_act * NUM_OBJ + o_ii
            score = h_cls * o_cls * iv
            plsc.store_scatter(pair_s, [kl], jnp.where(kmask, pair, -1))
            plsc.store_scatter(score_s, [kl], score)
            ch_cache.append((kl, kmask, iv, boxes, h_cls, o_cls, pair, score))

        segmaxes = [_splat_f(NEG_INF)] * 3
        for j in range(TOP_K):
            pj = plsc.load_gather(pair_s, [_splat_i(j)])
            sj = plsc.load_gather(score_s, [_splat_i(j)])
            for t in range(3):
                pr = ch_cache[t][6]
                segmaxes[t] = jnp.where(pr == pj,
                                        jnp.maximum(segmaxes[t], sj),
                                        segmaxes[t])
        for t in range(3):
            kl, kmask, iv, boxes, h_cls, o_cls, pair, score = ch_cache[t]
            valid = (iv > THRESH) & (h_cls > THRESH) & (o_cls > THRESH)
            final = jnp.where(valid & (score >= segmaxes[t]), score, 0.0)
            outch = boxes + [h_cls, o_cls, final]
            for c in range(NCHAN):
                plsc.store_scatter(out_v, [kl * NCHAN + c], outch[c],
                                   mask=kmask)

        pltpu.sync_copy(out_v, out_hbm.at[b])

    def batch_loop(i, _):
        b = wid + NW * i
        @pl.when(b < B)
        def _():
            do_batch(b)
        return 0
    lax.fori_loop(0, NBATCH_PER_W, batch_loop, 0)


def kernel(action_pred_logits, human_pred_logits, object_pred_logits,
           human_pred_boxes, object_pred_boxes, image_sizes):
    mesh = plsc.VectorSubcoreMesh(core_axis_name="c", subcore_axis_name="s",
                                  num_cores=2, num_subcores=16)
    f32 = jnp.float32
    out = pl.kernel(
        _body,
        out_type=jax.ShapeDtypeStruct((B, OUT_ROW), f32),
        mesh=mesh,
        compiler_params=pltpu.CompilerParams(needs_layout_passes=False,
                                             use_tc_tiling_on_sc=True),
        scratch_types=[
            pltpu.VMEM((Q * (NUM_ACT + 1),), f32),
            pltpu.VMEM((Q * 3,), f32),
            pltpu.VMEM((Q * (NUM_OBJ + 1),), f32),
            pltpu.VMEM((Q * 4,), f32),
            pltpu.VMEM((Q * 4,), f32),
            pltpu.VMEM((B * 2,), f32),
            pltpu.VMEM((NROW_PAD * L,), f32),
            pltpu.VMEM((NROW_PAD,), f32),
            pltpu.VMEM((3 * QPAD,), f32),
            pltpu.VMEM((3 * L,), f32),
            pltpu.VMEM((3 * L,), jnp.int32),
            pltpu.VMEM((3 * L,), jnp.int32),
            pltpu.VMEM((3 * L,), f32),
            pltpu.VMEM((L,), jnp.int32),
            pltpu.VMEM((OUT_ROW,), f32),
            pltpu.SemaphoreType.DMA((5,)),
        ],
    )(action_pred_logits.reshape(B, -1), human_pred_logits.reshape(B, -1),
      object_pred_logits.reshape(B, -1), human_pred_boxes.reshape(B, -1),
      object_pred_boxes.reshape(B, -1), image_sizes.reshape(-1))
    return out[:, :TOP_K * NCHAN].reshape(B, TOP_K, NCHAN)
